# baseline (speedup 1.0000x reference)
import functools

import jax
import jax.numpy as jnp
from jax import lax
from jax.experimental import pallas as pl
from jax.experimental.pallas import tpu as pltpu

_N, _T, _F = 1024, 512, 256
_B = 8
_NB = 8
_TB = 64
_NT = _T // _TB


def _body(nlen_ref, nblk_ref, x_ref, q_ref, o_ref):
    n = pl.program_id(0)
    t = pl.program_id(1)

    @pl.when(t == 0)
    def _init():
        o_ref[...] = jnp.zeros_like(o_ref)

    @pl.when(t < nblk_ref[n])
    def _step():
        q = q_ref[0]
        accs = []
        for k in range(_NB):
            node_len = nlen_ref[n * _NB + k]
            x = x_ref[k]
            s = jax.lax.dot_general(
                x, q.reshape(_F, 1),
                dimension_numbers=(((1,), (0,)), ((), ())),
                preferred_element_type=jnp.float32,
            )
            t_idx = lax.broadcasted_iota(jnp.int32, (_TB, 1), 0) + t * _TB
            w = jnp.where(t_idx < node_len, s, 0.0)
            acc = jax.lax.dot_general(
                w, x,
                dimension_numbers=(((0,), (0,)), ((), ())),
                preferred_element_type=jnp.float32,
            )
            accs.append(acc)
        o_ref[...] += jnp.concatenate(accs, axis=0)


def kernel(nodes_output, ptr, lengths, Wq_w):
    ptr_i = ptr.astype(jnp.int32)
    len_i = lengths.astype(jnp.int32)
    num_nodes = ptr_i[1:] - ptr_i[:-1]
    node_len = jnp.repeat(len_i, num_nodes, total_repeat_length=_N)
    grp_max = jnp.max(node_len.reshape(_N // _NB, _NB), axis=1)
    grp_nblk = (grp_max + (_TB - 1)) // _TB
    q2 = Wq_w.reshape(1, _F)

    grid_spec = pltpu.PrefetchScalarGridSpec(
        num_scalar_prefetch=2,
        grid=(_N // _NB, _NT),
        in_specs=[
            pl.BlockSpec(
                (_NB, _TB, _F),
                lambda n, t, nlen, nblk: (n, jnp.minimum(t, nblk[n] - 1), 0),
            ),
            pl.BlockSpec((1, _F), lambda n, t, nlen, nblk: (0, 0)),
        ],
        out_specs=pl.BlockSpec((_NB, _F), lambda n, t, nlen, nblk: (n, 0)),
    )
    return pl.pallas_call(
        _body,
        grid_spec=grid_spec,
        out_shape=jax.ShapeDtypeStruct((_N, _F), jnp.float32),
    )(node_len, grp_nblk, nodes_output, q2)


# speedup vs baseline: 1.9554x; 1.9554x over previous
import functools

import jax
import jax.numpy as jnp
from jax import lax
from jax.experimental import pallas as pl
from jax.experimental.pallas import tpu as pltpu

_N, _T, _F = 1024, 512, 256
_B = 8
_NB = 8
_TB = 128
_NT = _T // _TB


def _body(nlen_ref, nblk_ref, x_ref, q_ref, o_ref):
    n = pl.program_id(0)
    t = pl.program_id(1)

    @pl.when(t == 0)
    def _init():
        o_ref[...] = jnp.zeros_like(o_ref)

    @pl.when(t < nblk_ref[n])
    def _step():
        q = q_ref[0]
        x2 = x_ref[...].reshape(_NB * _TB, _F)
        s = jax.lax.dot_general(
            x2, q.reshape(_F, 1),
            dimension_numbers=(((1,), (0,)), ((), ())),
            preferred_element_type=jnp.float32,
        )
        col = lax.broadcasted_iota(jnp.int32, (_NB, _NB * _TB), 1)
        row = lax.broadcasted_iota(jnp.int32, (_NB, _NB * _TB), 0)
        lens = jnp.concatenate(
            [jnp.full((1, _NB * _TB), nlen_ref[n * _NB + k], jnp.int32)
             for k in range(_NB)], axis=0)
        t_glob = (col - row * _TB) + t * _TB
        valid = (col // _TB == row) & (t_glob < lens)
        w_bd = jnp.where(valid, s.reshape(1, _NB * _TB), 0.0)
        o_ref[...] += jax.lax.dot_general(
            w_bd, x2,
            dimension_numbers=(((1,), (0,)), ((), ())),
            preferred_element_type=jnp.float32,
        )


def kernel(nodes_output, ptr, lengths, Wq_w):
    ptr_i = ptr.astype(jnp.int32)
    len_i = lengths.astype(jnp.int32)
    num_nodes = ptr_i[1:] - ptr_i[:-1]
    node_len = jnp.repeat(len_i, num_nodes, total_repeat_length=_N)
    grp_max = jnp.max(node_len.reshape(_N // _NB, _NB), axis=1)
    grp_nblk = (grp_max + (_TB - 1)) // _TB
    q2 = Wq_w.reshape(1, _F)

    grid_spec = pltpu.PrefetchScalarGridSpec(
        num_scalar_prefetch=2,
        grid=(_N // _NB, _NT),
        in_specs=[
            pl.BlockSpec(
                (_NB, _TB, _F),
                lambda n, t, nlen, nblk: (n, jnp.minimum(t, nblk[n] - 1), 0),
            ),
            pl.BlockSpec((1, _F), lambda n, t, nlen, nblk: (0, 0)),
        ],
        out_specs=pl.BlockSpec((_NB, _F), lambda n, t, nlen, nblk: (n, 0)),
    )
    return pl.pallas_call(
        _body,
        grid_spec=grid_spec,
        out_shape=jax.ShapeDtypeStruct((_N, _F), jnp.float32),
    )(node_len, grp_nblk, nodes_output, q2)


# speedup vs baseline: 3.6553x; 1.8694x over previous
import functools

import jax
import jax.numpy as jnp
from jax import lax
from jax.experimental import pallas as pl
from jax.experimental.pallas import tpu as pltpu

_N, _T, _F = 1024, 512, 256
_B = 8
_NB = 8
_TB = 512
_NT = _T // _TB


def _body(nlen_ref, nblk_ref, x_ref, q_ref, o_ref):
    n = pl.program_id(0)
    t = pl.program_id(1)

    @pl.when(t == 0)
    def _init():
        o_ref[...] = jnp.zeros_like(o_ref)

    @pl.when(t < nblk_ref[n])
    def _step():
        q = q_ref[0]
        x2 = x_ref[...].reshape(_NB * _TB, _F)
        s = jax.lax.dot_general(
            x2, q.reshape(_F, 1),
            dimension_numbers=(((1,), (0,)), ((), ())),
            preferred_element_type=jnp.float32,
        )
        col = lax.broadcasted_iota(jnp.int32, (_NB, _NB * _TB), 1)
        row = lax.broadcasted_iota(jnp.int32, (_NB, _NB * _TB), 0)
        lens = jnp.concatenate(
            [jnp.full((1, _NB * _TB), nlen_ref[n * _NB + k], jnp.int32)
             for k in range(_NB)], axis=0)
        t_glob = (col - row * _TB) + t * _TB
        valid = (col // _TB == row) & (t_glob < lens)
        w_bd = jnp.where(valid, s.reshape(1, _NB * _TB), 0.0)
        o_ref[...] += jax.lax.dot_general(
            w_bd, x2,
            dimension_numbers=(((1,), (0,)), ((), ())),
            preferred_element_type=jnp.float32,
        )


def kernel(nodes_output, ptr, lengths, Wq_w):
    ptr_i = ptr.astype(jnp.int32)
    len_i = lengths.astype(jnp.int32)
    num_nodes = ptr_i[1:] - ptr_i[:-1]
    node_len = jnp.repeat(len_i, num_nodes, total_repeat_length=_N)
    grp_max = jnp.max(node_len.reshape(_N // _NB, _NB), axis=1)
    grp_nblk = (grp_max + (_TB - 1)) // _TB
    q2 = Wq_w.reshape(1, _F)

    grid_spec = pltpu.PrefetchScalarGridSpec(
        num_scalar_prefetch=2,
        grid=(_N // _NB, _NT),
        in_specs=[
            pl.BlockSpec(
                (_NB, _TB, _F),
                lambda n, t, nlen, nblk: (n, jnp.minimum(t, nblk[n] - 1), 0),
            ),
            pl.BlockSpec((1, _F), lambda n, t, nlen, nblk: (0, 0)),
        ],
        out_specs=pl.BlockSpec((_NB, _F), lambda n, t, nlen, nblk: (n, 0)),
    )
    return pl.pallas_call(
        _body,
        grid_spec=grid_spec,
        out_shape=jax.ShapeDtypeStruct((_N, _F), jnp.float32),
    )(node_len, grp_nblk, nodes_output, q2)


# speedup vs baseline: 4.8734x; 1.3332x over previous
import functools

import jax
import jax.numpy as jnp
from jax import lax
from jax.experimental import pallas as pl
from jax.experimental.pallas import tpu as pltpu

_N, _T, _F = 1024, 512, 256
_B = 8
_NB = 64
_TB = 64
_NT = _T // _TB


def _body(nlen_ref, nblk_ref, x_ref, q_ref, o_ref):
    n = pl.program_id(0)
    t = pl.program_id(1)

    @pl.when(t == 0)
    def _init():
        o_ref[...] = jnp.zeros_like(o_ref)

    @pl.when(t < nblk_ref[n])
    def _step():
        q = q_ref[0]
        x2 = x_ref[...].reshape(_NB * _TB, _F)
        s = jax.lax.dot_general(
            x2, q.reshape(_F, 1),
            dimension_numbers=(((1,), (0,)), ((), ())),
            preferred_element_type=jnp.float32,
        )
        col = lax.broadcasted_iota(jnp.int32, (_NB, _NB * _TB), 1)
        row = lax.broadcasted_iota(jnp.int32, (_NB, _NB * _TB), 0)
        lens = jnp.concatenate(
            [jnp.full((1, _NB * _TB), nlen_ref[n * _NB + k], jnp.int32)
             for k in range(_NB)], axis=0)
        t_glob = (col - row * _TB) + t * _TB
        valid = (col // _TB == row) & (t_glob < lens)
        w_bd = jnp.where(valid, s.reshape(1, _NB * _TB), 0.0)
        o_ref[...] += jax.lax.dot_general(
            w_bd, x2,
            dimension_numbers=(((1,), (0,)), ((), ())),
            preferred_element_type=jnp.float32,
        )


def kernel(nodes_output, ptr, lengths, Wq_w):
    ptr_i = ptr.astype(jnp.int32)
    len_i = lengths.astype(jnp.int32)
    num_nodes = ptr_i[1:] - ptr_i[:-1]
    node_len = jnp.repeat(len_i, num_nodes, total_repeat_length=_N)
    grp_max = jnp.max(node_len.reshape(_N // _NB, _NB), axis=1)
    grp_nblk = (grp_max + (_TB - 1)) // _TB
    q2 = Wq_w.reshape(1, _F)

    grid_spec = pltpu.PrefetchScalarGridSpec(
        num_scalar_prefetch=2,
        grid=(_N // _NB, _NT),
        in_specs=[
            pl.BlockSpec(
                (_NB, _TB, _F),
                lambda n, t, nlen, nblk: (n, jnp.minimum(t, nblk[n] - 1), 0),
            ),
            pl.BlockSpec((1, _F), lambda n, t, nlen, nblk: (0, 0)),
        ],
        out_specs=pl.BlockSpec((_NB, _F), lambda n, t, nlen, nblk: (n, 0)),
    )
    return pl.pallas_call(
        _body,
        grid_spec=grid_spec,
        out_shape=jax.ShapeDtypeStruct((_N, _F), jnp.float32),
    )(node_len, grp_nblk, nodes_output, q2)


# speedup vs baseline: 4.9405x; 1.0138x over previous
import functools

import jax
import jax.numpy as jnp
from jax import lax
from jax.experimental import pallas as pl
from jax.experimental.pallas import tpu as pltpu

_N, _T, _F = 1024, 512, 256
_B = 8
_NB = 64
_TB = 64
_NT = _T // _TB


def _body(nlen_ref, nblk_ref, x_ref, q_ref, o_ref):
    n = pl.program_id(0)
    t = pl.program_id(1)

    @pl.when(t == 0)
    def _init():
        o_ref[...] = jnp.zeros_like(o_ref)

    @pl.when(t < nblk_ref[n])
    def _step():
        q = q_ref[0]
        x2 = x_ref[...].reshape(_NB * _TB, _F)
        s = jax.lax.dot_general(
            x2, q.reshape(_F, 1),
            dimension_numbers=(((1,), (0,)), ((), ())),
            preferred_element_type=jnp.float32,
        )
        sg = 8
        cols = sg * _TB
        col = lax.broadcasted_iota(jnp.int32, (sg, cols), 1)
        row = lax.broadcasted_iota(jnp.int32, (sg, cols), 0)
        diag = col // _TB == row
        t_loc = (col - row * _TB) + t * _TB
        s2 = s.reshape(_NB // sg, 1, cols)
        for j in range(_NB // sg):
            lens = jnp.concatenate(
                [jnp.full((1, cols), nlen_ref[n * _NB + j * sg + k], jnp.int32)
                 for k in range(sg)], axis=0)
            w_bd = jnp.where(diag & (t_loc < lens), s2[j], 0.0)
            o_ref[j * sg:(j + 1) * sg, :] += jax.lax.dot_general(
                w_bd, x_ref[j * sg:(j + 1) * sg].reshape(cols, _F),
                dimension_numbers=(((1,), (0,)), ((), ())),
                preferred_element_type=jnp.float32,
            )


def kernel(nodes_output, ptr, lengths, Wq_w):
    ptr_i = ptr.astype(jnp.int32)
    len_i = lengths.astype(jnp.int32)
    num_nodes = ptr_i[1:] - ptr_i[:-1]
    node_len = jnp.repeat(len_i, num_nodes, total_repeat_length=_N)
    grp_max = jnp.max(node_len.reshape(_N // _NB, _NB), axis=1)
    grp_nblk = (grp_max + (_TB - 1)) // _TB
    q2 = Wq_w.reshape(1, _F)

    grid_spec = pltpu.PrefetchScalarGridSpec(
        num_scalar_prefetch=2,
        grid=(_N // _NB, _NT),
        in_specs=[
            pl.BlockSpec(
                (_NB, _TB, _F),
                lambda n, t, nlen, nblk: (n, jnp.minimum(t, nblk[n] - 1), 0),
            ),
            pl.BlockSpec((1, _F), lambda n, t, nlen, nblk: (0, 0)),
        ],
        out_specs=pl.BlockSpec((_NB, _F), lambda n, t, nlen, nblk: (n, 0)),
    )
    return pl.pallas_call(
        _body,
        grid_spec=grid_spec,
        out_shape=jax.ShapeDtypeStruct((_N, _F), jnp.float32),
    )(node_len, grp_nblk, nodes_output, q2)
